# baseline (speedup 1.0000x reference)
import jax
import jax.numpy as jnp
from jax import lax
from jax.experimental import pallas as pl
from jax.experimental.pallas import tpu as pltpu
from jax.experimental.pallas import tpu_sc as plsc


_ROWS = 8192
_COLS = 1024
_NC = 2
_NS = 16
_HALF = _ROWS // _NC
_ROWS_PER_W = _HALF // _NS
_CHUNK = 32
_N_CHUNKS = _ROWS_PER_W // _CHUNK


def _sc_copy(x_hbm, out0_hbm, out1_hbm, buf, lsem, ssem):
    cid = lax.axis_index("c")
    sid = lax.axis_index("s")
    base = cid * _HALF + sid * _ROWS_PER_W
    out_base = sid * _ROWS_PER_W

    def load(i):
        return pltpu.make_async_copy(
            x_hbm.at[pl.ds(base + i * _CHUNK, _CHUNK), :],
            buf.at[i % 2],
            lsem.at[i % 2],
        )

    def store(out_hbm, i):
        return pltpu.make_async_copy(
            buf.at[i % 2],
            out_hbm.at[pl.ds(out_base + i * _CHUNK, _CHUNK), :],
            ssem.at[i % 2],
        )

    def run(out_hbm):
        load(0).start()
        for i in range(_N_CHUNKS):
            if i + 1 < _N_CHUNKS:
                if i - 1 >= 0:
                    store(out_hbm, i - 1).wait()
                load(i + 1).start()
            load(i).wait()
            store(out_hbm, i).start()
        store(out_hbm, _N_CHUNKS - 2).wait()
        store(out_hbm, _N_CHUNKS - 1).wait()

    @pl.when(cid == 0)
    def _():
        run(out0_hbm)

    @pl.when(cid == 1)
    def _():
        run(out1_hbm)


def kernel(x):
    mesh = plsc.VectorSubcoreMesh(core_axis_name="c", subcore_axis_name="s")
    half = jax.ShapeDtypeStruct((_HALF, _COLS), jnp.float32)
    out0, out1 = pl.kernel(
        _sc_copy,
        out_type=[half, half],
        mesh=mesh,
        scratch_types=[
            pltpu.VMEM((2, _CHUNK, _COLS), jnp.float32),
            pltpu.SemaphoreType.DMA((2,)),
            pltpu.SemaphoreType.DMA((2,)),
        ],
    )(x)
    gathered = jnp.concatenate([out0, out1], axis=0)
    sizes = jnp.array([_ROWS], dtype=jnp.int32)
    return (gathered, sizes)


# speedup vs baseline: 1.8975x; 1.8975x over previous
import jax
import jax.numpy as jnp
from jax import lax
from jax.experimental import pallas as pl
from jax.experimental.pallas import tpu as pltpu
from jax.experimental.pallas import tpu_sc as plsc


_ROWS = 8192
_COLS = 1024
_BLOCK_ROWS = 2048


def _copy_kernel(x_ref, o_ref):
    o_ref[...] = x_ref[...]


def _sc_sizes(sizes_hbm, vec, sem):
    sid = lax.axis_index("s")

    @pl.when(sid == 0)
    def _():
        vec[...] = jnp.full((16,), _ROWS, dtype=jnp.int32)
        pltpu.make_async_copy(vec.at[pl.ds(0, 1)], sizes_hbm, sem).start()
        pltpu.make_async_copy(vec.at[pl.ds(0, 1)], sizes_hbm, sem).wait()


def kernel(x):
    n_blocks = _ROWS // _BLOCK_ROWS
    gathered = pl.pallas_call(
        _copy_kernel,
        grid=(n_blocks,),
        in_specs=[pl.BlockSpec((_BLOCK_ROWS, _COLS), lambda i: (i, 0))],
        out_specs=pl.BlockSpec((_BLOCK_ROWS, _COLS), lambda i: (i, 0)),
        out_shape=jax.ShapeDtypeStruct((_ROWS, _COLS), x.dtype),
        compiler_params=pltpu.CompilerParams(
            dimension_semantics=("parallel",),
        ),
    )(x)
    sc_mesh = plsc.VectorSubcoreMesh(
        core_axis_name="c", subcore_axis_name="s", num_cores=1
    )
    sizes = pl.kernel(
        _sc_sizes,
        out_type=jax.ShapeDtypeStruct((1,), jnp.int32),
        mesh=sc_mesh,
        scratch_types=[
            pltpu.VMEM((16,), jnp.int32),
            pltpu.SemaphoreType.DMA,
        ],
    )()
    return (gathered, sizes)


# speedup vs baseline: 2.8013x; 1.4763x over previous
import jax
import jax.numpy as jnp
from jax.experimental import pallas as pl
from jax.experimental.pallas import tpu as pltpu


_ROWS = 8192
_COLS = 1024
_BLOCK_ROWS = 1024


def _copy_kernel(x_ref, o_ref):
    o_ref[...] = x_ref[...]


def kernel(x):
    n_blocks = _ROWS // _BLOCK_ROWS
    gathered = pl.pallas_call(
        _copy_kernel,
        grid=(n_blocks,),
        in_specs=[pl.BlockSpec((_BLOCK_ROWS, _COLS), lambda i: (i, 0))],
        out_specs=pl.BlockSpec((_BLOCK_ROWS, _COLS), lambda i: (i, 0)),
        out_shape=jax.ShapeDtypeStruct((_ROWS, _COLS), x.dtype),
        compiler_params=pltpu.CompilerParams(
            dimension_semantics=("parallel",),
        ),
    )(x)
    sizes = jnp.array([_ROWS], dtype=jnp.int32)
    return (gathered, sizes)
